# baseline (speedup 1.0000x reference)
import functools

import jax
import jax.numpy as jnp
from jax import lax
from jax.experimental import pallas as pl
from jax.experimental.pallas import tpu as pltpu
from jax.experimental.pallas import tpu_sc as plsc

D = 128
N = 10000
E = 320000
TEMP = float(D) ** 0.5

NC = 2
NS = 16
NW = NC * NS
EPW = E // NW
CH = 80
NCH = EPW // CH

ZR = 40
NZCT = N // ZR
NP1 = 10240


def _mesh():
    return plsc.VectorSubcoreMesh(core_axis_name="c", subcore_axis_name="s")


K = 5
UNITS = [10, 30, 30, 30, 25]
CHUNK_E = [u * NW * CH for u in UNITS]
CHUNK_B = [NW * CH * sum(UNITS[:i]) for i in range(K)]


def _gather_body(eb, npw, nch, x_hbm, src_hbm, g_hbm, idx_v, rows0, rows1,
                 gsem0, gsem1, wsem0, wsem1):
    c = lax.axis_index("c")
    s = lax.axis_index("s")
    wid = s * NC + c
    base = eb + wid * npw
    pltpu.sync_copy(src_hbm.at[pl.ds(base, npw)], idx_v)
    rows = (rows0, rows1)
    gsem = (gsem0, gsem1)
    wsem = (wsem0, wsem1)

    def gstart(j, b):
        pltpu.async_copy(x_hbm.at[idx_v.at[pl.ds(j * CH, CH)]], rows[b], gsem[b])

    def gwait(b):
        pltpu.make_async_copy(x_hbm.at[pl.ds(0, CH)], rows[b], gsem[b]).wait()

    def wstart(j, b):
        pltpu.async_copy(rows[b], g_hbm.at[pl.ds(wid * npw + j * CH, CH)], wsem[b])

    def wwait(j, b):
        pltpu.make_async_copy(rows[b], g_hbm.at[pl.ds(wid * npw + j * CH, CH)],
                              wsem[b]).wait()

    for j in range(nch):
        b = j & 1
        if j >= 2:
            wwait(j - 2, b)
        gstart(j, b)
        if j >= 1:
            gwait(1 - b)
            wstart(j - 1, 1 - b)
    bl = (nch - 1) & 1
    gwait(bl)
    wstart(nch - 1, bl)
    if nch >= 2:
        wwait(nch - 2, 1 - bl)
    wwait(nch - 1, bl)


def _sc_gather(x, src, kc):
    ec = CHUNK_E[kc]
    npw = ec // NW
    k = pl.kernel(
        functools.partial(_gather_body, CHUNK_B[kc], npw, npw // CH),
        out_type=jax.ShapeDtypeStruct((ec, D), jnp.float32),
        mesh=_mesh(),
        scratch_types=[
            pltpu.VMEM((npw,), jnp.int32),
            pltpu.VMEM((CH, D), jnp.float32),
            pltpu.VMEM((CH, D), jnp.float32),
            pltpu.SemaphoreType.DMA,
            pltpu.SemaphoreType.DMA,
            pltpu.SemaphoreType.DMA,
            pltpu.SemaphoreType.DMA,
        ],
    )
    return k(x, src)


EB = 1280
NEB = E // EB
WPR = EB // 128


def _edge_body(g_ref, eh_ref, qrh_ref, qeh_ref, wmk_ref, wqt_ref, wmsg_ref, wp_ref):
    g = g_ref[...]
    eh = eh_ref[...]
    mk = (jnp.dot(g, wmk_ref[:D], preferred_element_type=jnp.float32)
          + jnp.dot(eh, wmk_ref[D:], preferred_element_type=jnp.float32))
    q = (jnp.dot(qrh_ref[...], wqt_ref[:D], preferred_element_type=jnp.float32)
         + jnp.dot(qeh_ref[...], wqt_ref[D:], preferred_element_type=jnp.float32))
    m = mk[:, :D]
    msg = jnp.where(m >= 0, m, 0.01 * m)
    k = mk[:, D:]
    att = jnp.sum(q * k, axis=-1, keepdims=True)
    w = jnp.exp(att)
    wmsg_ref[...] = w * msg
    e_i = lax.broadcasted_iota(jnp.int32, (EB, 128), 0)
    l_i = lax.broadcasted_iota(jnp.int32, (EB, 128), 1)
    B = (e_i % 128 == l_i).astype(jnp.float32)
    g_i = lax.broadcasted_iota(jnp.int32, (WPR, EB), 0)
    e2_i = lax.broadcasted_iota(jnp.int32, (WPR, EB), 1)
    A = (e2_i // 128 == g_i).astype(jnp.float32)
    wp_ref[0] = jnp.dot(A, w * B, preferred_element_type=jnp.float32)


def _tc_edge(g, edge_h, edge_qrh, edge_qeh, wmk, wqt, kc):
    gec = CHUNK_E[kc] // EB
    off = CHUNK_B[kc] // EB
    return pl.pallas_call(
        _edge_body,
        grid=(gec,),
        in_specs=[
            pl.BlockSpec((EB, D), lambda i: (i, 0)),
            pl.BlockSpec((EB, D), lambda i: (i + off, 0)),
            pl.BlockSpec((EB, D), lambda i: (i + off, 0)),
            pl.BlockSpec((EB, D), lambda i: (i + off, 0)),
            pl.BlockSpec((2 * D, 2 * D), lambda i: (0, 0)),
            pl.BlockSpec((2 * D, D), lambda i: (0, 0)),
        ],
        out_specs=[
            pl.BlockSpec((EB, D), lambda i: (i, 0)),
            pl.BlockSpec((1, WPR, 128), lambda i: (i, 0, 0)),
        ],
        out_shape=[
            jax.ShapeDtypeStruct((CHUNK_E[kc], D), jnp.float32),
            jax.ShapeDtypeStruct((gec, WPR, 128), jnp.float32),
        ],
    )(g, edge_h, edge_qrh, edge_qeh, wmk, wqt)


def _scatter_body(eb, npw, nch, wmsg_hbm, w_hbm, dst_hbm, nump_hbm, denp_hbm,
                  ix0, ix1, wm0, wm1, wv0, wv1, zb_v, zb1_v, num_sh, den_sh,
                  lsem0, lsem1, ssem0, ssem1, zsem):
    c = lax.axis_index("c")
    s = lax.axis_index("s")
    wid = s * NC + c
    base = wid * npw
    dbase = eb + base
    ix = (ix0, ix1)
    wm = (wm0, wm1)
    wv = (wv0, wv1)
    lsem = (lsem0, lsem1)
    ssem = (ssem0, ssem1)

    def lstart(j, b):
        pltpu.async_copy(dst_hbm.at[pl.ds(dbase + j * CH, CH)], ix[b], lsem[b])
        pltpu.async_copy(wmsg_hbm.at[pl.ds(base + j * CH, CH)], wm[b], lsem[b])
        pltpu.async_copy(w_hbm.at[pl.ds(base + j * CH, CH)], wv[b], lsem[b])

    def lwait(j, b):
        pltpu.make_async_copy(dst_hbm.at[pl.ds(dbase + j * CH, CH)], ix[b], lsem[b]).wait()
        pltpu.make_async_copy(wmsg_hbm.at[pl.ds(base + j * CH, CH)], wm[b], lsem[b]).wait()
        pltpu.make_async_copy(w_hbm.at[pl.ds(base + j * CH, CH)], wv[b], lsem[b]).wait()

    lstart(0, 0)

    def zrow(i, carry):
        for l in range(D // 16):
            zb_v[i, pl.ds(l * 16, 16)] = jnp.zeros((16,), jnp.float32)
        return carry

    lax.fori_loop(0, ZR, zrow, 0)

    def zrow1(i, carry):
        zb1_v[pl.ds(i * 16, 16)] = jnp.zeros((16,), jnp.float32)
        return carry

    lax.fori_loop(0, 40, zrow1, 0)

    def zc(i, carry):
        cc = s + i * NS

        @pl.when(cc < NZCT)
        def _():
            pltpu.sync_copy(zb_v, num_sh.at[pl.ds(cc * ZR, ZR)])

        return carry

    lax.fori_loop(0, 16, zc, 0)
    pltpu.sync_copy(zb1_v, den_sh.at[pl.ds(s * 640, 640)])
    plsc.subcore_barrier()

    def sstart(j, b):
        pltpu.async_copy(wm[b], num_sh.at[ix[b]], ssem[b], add=True)
        pltpu.async_copy(wv[b], den_sh.at[ix[b]], ssem[b], add=True)

    def swait(b):
        pltpu.make_async_copy(wm[b], num_sh.at[pl.ds(0, CH)], ssem[b]).wait()
        pltpu.make_async_copy(wv[b], den_sh.at[pl.ds(0, CH)], ssem[b]).wait()

    for j in range(nch):
        b = j & 1
        lwait(j, b)
        if j >= 1:
            swait(1 - b)
        if j + 1 < nch:
            lstart(j + 1, 1 - b)
        sstart(j, b)
    swait((nch - 1) & 1)
    plsc.subcore_barrier()

    def rc(i, carry):
        cc = s + i * NS

        @pl.when(cc < NZCT)
        def _():
            pltpu.sync_copy(num_sh.at[pl.ds(cc * ZR, ZR)], zb_v)
            pltpu.sync_copy(zb_v, nump_hbm.at[c, pl.ds(cc * ZR, ZR)])

        return carry

    lax.fori_loop(0, 16, rc, 0)
    pltpu.sync_copy(den_sh.at[pl.ds(s * 640, 640)], zb1_v)
    pltpu.sync_copy(zb1_v, denp_hbm.at[c, pl.ds(s * 640, 640)])


def _sc_scatter(wmsg, w, dst, kc):
    npw = CHUNK_E[kc] // NW
    k = pl.kernel(
        functools.partial(_scatter_body, CHUNK_B[kc], npw, npw // CH),
        out_type=(
            jax.ShapeDtypeStruct((NC, N, D), jnp.float32),
            jax.ShapeDtypeStruct((NC, NP1), jnp.float32),
        ),
        mesh=_mesh(),
        scratch_types=[
            pltpu.VMEM((CH,), jnp.int32),
            pltpu.VMEM((CH,), jnp.int32),
            pltpu.VMEM((CH, D), jnp.float32),
            pltpu.VMEM((CH, D), jnp.float32),
            pltpu.VMEM((CH,), jnp.float32),
            pltpu.VMEM((CH,), jnp.float32),
            pltpu.VMEM((ZR, D), jnp.float32),
            pltpu.VMEM((640,), jnp.float32),
            pltpu.VMEM_SHARED((N, D), jnp.float32),
            pltpu.VMEM_SHARED((NP1,), jnp.float32),
            pltpu.SemaphoreType.DMA,
            pltpu.SemaphoreType.DMA,
            pltpu.SemaphoreType.DMA,
            pltpu.SemaphoreType.DMA,
            pltpu.SemaphoreType.DMA,
        ],
    )
    return k(wmsg, w, dst)


NB = 1000


def _final_body(np0, np1, np2, np3, np4, den_ref, x_ref, wout_ref, gamma_ref, beta_ref, out_ref):
    num = (np0[0] + np0[1] + np1[0] + np1[1] + np2[0] + np2[1]
           + np3[0] + np3[1] + np4[0] + np4[1])
    den = den_ref[...]
    agg = num * jnp.where(den > 0, 1.0 / jnp.where(den > 0, den, 1.0), 0.0)
    t = jnp.dot(agg, wout_ref[...], preferred_element_type=jnp.float32)
    h = jnp.where(t >= 0, t, 0.01 * t) + x_ref[...]
    mu = jnp.mean(h, axis=-1, keepdims=True)
    var = jnp.mean((h - mu) ** 2, axis=-1, keepdims=True)
    out_ref[...] = (h - mu) * lax.rsqrt(var + 1e-6) * gamma_ref[...] + beta_ref[...]


def _tc_final(numps, den, x, wout, gamma, beta):
    return pl.pallas_call(
        _final_body,
        grid=(N // NB,),
        in_specs=[
            pl.BlockSpec((NC, NB, D), lambda i: (0, i, 0)),
            pl.BlockSpec((NC, NB, D), lambda i: (0, i, 0)),
            pl.BlockSpec((NC, NB, D), lambda i: (0, i, 0)),
            pl.BlockSpec((NC, NB, D), lambda i: (0, i, 0)),
            pl.BlockSpec((NC, NB, D), lambda i: (0, i, 0)),
            pl.BlockSpec((NB, 1), lambda i: (i, 0)),
            pl.BlockSpec((NB, D), lambda i: (i, 0)),
            pl.BlockSpec((D, D), lambda i: (0, 0)),
            pl.BlockSpec((1, D), lambda i: (0, 0)),
            pl.BlockSpec((1, D), lambda i: (0, 0)),
        ],
        out_specs=pl.BlockSpec((NB, D), lambda i: (i, 0)),
        out_shape=jax.ShapeDtypeStruct((N, D), jnp.float32),
    )(*numps, den, x, wout, gamma, beta)


def kernel(x, edge_h, edge_qrh, edge_qeh, W_msg, W_q, W_k, W_out, ln_gamma, ln_beta, edge_index):
    src = edge_index[0].astype(jnp.int32)
    dst = edge_index[1].astype(jnp.int32)
    wmk = jnp.concatenate([W_msg.T, W_k.T], axis=1)
    wqt = W_q.T * (1.0 / TEMP)

    numps = []
    den = jnp.zeros((N,), jnp.float32)
    for kc in range(K):
        g = _sc_gather(x, src, kc)
        wmsg, wp = _tc_edge(g, edge_h, edge_qrh, edge_qeh, wmk, wqt, kc)
        w = wp.reshape(CHUNK_E[kc])
        nump, denp = _sc_scatter(wmsg, w, dst, kc)
        numps.append(nump)
        den = den + denp[0, :N] + denp[1, :N]
    return _tc_final(numps, den.reshape(N, 1), x, W_out.T,
                     ln_gamma.reshape(1, D), ln_beta.reshape(1, D))


# speedup vs baseline: 1.0186x; 1.0186x over previous
import functools

import jax
import jax.numpy as jnp
from jax import lax
from jax.experimental import pallas as pl
from jax.experimental.pallas import tpu as pltpu
from jax.experimental.pallas import tpu_sc as plsc

D = 128
N = 10000
E = 320000
TEMP = float(D) ** 0.5

NC = 2
NS = 16
NW = NC * NS
EPW = E // NW
CH = 80
NCH = EPW // CH

ZR = 40
NZCT = N // ZR
NP1 = 10240


def _mesh():
    return plsc.VectorSubcoreMesh(core_axis_name="c", subcore_axis_name="s")


K = 5
UNITS = [10, 30, 30, 30, 25]
CHUNK_E = [u * NW * CH for u in UNITS]
CHUNK_B = [NW * CH * sum(UNITS[:i]) for i in range(K)]


def _gather_body(eb, npw, nch, x_hbm, src_hbm, g_hbm, idx_v, rows0, rows1,
                 gsem0, gsem1, wsem0, wsem1):
    c = lax.axis_index("c")
    s = lax.axis_index("s")
    wid = s * NC + c
    base = eb + wid * npw
    pltpu.sync_copy(src_hbm.at[pl.ds(base, npw)], idx_v)
    rows = (rows0, rows1)
    gsem = (gsem0, gsem1)
    wsem = (wsem0, wsem1)

    def gstart(j, b):
        pltpu.async_copy(x_hbm.at[idx_v.at[pl.ds(j * CH, CH)]], rows[b], gsem[b])

    def gwait(b):
        pltpu.make_async_copy(x_hbm.at[pl.ds(0, CH)], rows[b], gsem[b]).wait()

    def wstart(j, b):
        pltpu.async_copy(rows[b], g_hbm.at[pl.ds(wid * npw + j * CH, CH)], wsem[b])

    def wwait(j, b):
        pltpu.make_async_copy(rows[b], g_hbm.at[pl.ds(wid * npw + j * CH, CH)],
                              wsem[b]).wait()

    for j in range(nch):
        b = j & 1
        if j >= 2:
            wwait(j - 2, b)
        gstart(j, b)
        if j >= 1:
            gwait(1 - b)
            wstart(j - 1, 1 - b)
    bl = (nch - 1) & 1
    gwait(bl)
    wstart(nch - 1, bl)
    if nch >= 2:
        wwait(nch - 2, 1 - bl)
    wwait(nch - 1, bl)


def _sc_gather(x, src, kc):
    ec = CHUNK_E[kc]
    npw = ec // NW
    k = pl.kernel(
        functools.partial(_gather_body, CHUNK_B[kc], npw, npw // CH),
        out_type=jax.ShapeDtypeStruct((ec, D), jnp.float32),
        mesh=_mesh(),
        scratch_types=[
            pltpu.VMEM((npw,), jnp.int32),
            pltpu.VMEM((CH, D), jnp.float32),
            pltpu.VMEM((CH, D), jnp.float32),
            pltpu.SemaphoreType.DMA,
            pltpu.SemaphoreType.DMA,
            pltpu.SemaphoreType.DMA,
            pltpu.SemaphoreType.DMA,
        ],
    )
    return k(x, src)


EB = 1280
NEB = E // EB
WPR = EB // 128


def _edge_body(g_ref, eh_ref, qrh_ref, qeh_ref, wmk_ref, wqt_ref, wmsg_ref, wp_ref):
    g = g_ref[...]
    eh = eh_ref[...]
    mk = (jnp.dot(g, wmk_ref[:D], preferred_element_type=jnp.float32)
          + jnp.dot(eh, wmk_ref[D:], preferred_element_type=jnp.float32))
    q = (jnp.dot(qrh_ref[...], wqt_ref[:D], preferred_element_type=jnp.float32)
         + jnp.dot(qeh_ref[...], wqt_ref[D:], preferred_element_type=jnp.float32))
    m = mk[:, :D]
    msg = jnp.where(m >= 0, m, 0.01 * m)
    k = mk[:, D:]
    att = jnp.sum(q * k, axis=-1, keepdims=True)
    w = jnp.exp(att)
    wmsg_ref[...] = w * msg
    e_i = lax.broadcasted_iota(jnp.int32, (EB, 128), 0)
    l_i = lax.broadcasted_iota(jnp.int32, (EB, 128), 1)
    B = (e_i % 128 == l_i).astype(jnp.float32)
    g_i = lax.broadcasted_iota(jnp.int32, (WPR, EB), 0)
    e2_i = lax.broadcasted_iota(jnp.int32, (WPR, EB), 1)
    A = (e2_i // 128 == g_i).astype(jnp.float32)
    wp_ref[0] = jnp.dot(A, w * B, preferred_element_type=jnp.float32)


def _tc_edge(g, edge_h, edge_qrh, edge_qeh, wmk, wqt, kc):
    gec = CHUNK_E[kc] // EB
    off = CHUNK_B[kc] // EB
    return pl.pallas_call(
        _edge_body,
        grid=(gec,),
        in_specs=[
            pl.BlockSpec((EB, D), lambda i: (i, 0)),
            pl.BlockSpec((EB, D), lambda i: (i + off, 0)),
            pl.BlockSpec((EB, D), lambda i: (i + off, 0)),
            pl.BlockSpec((EB, D), lambda i: (i + off, 0)),
            pl.BlockSpec((2 * D, 2 * D), lambda i: (0, 0)),
            pl.BlockSpec((2 * D, D), lambda i: (0, 0)),
        ],
        out_specs=[
            pl.BlockSpec((EB, D), lambda i: (i, 0)),
            pl.BlockSpec((1, WPR, 128), lambda i: (i, 0, 0)),
        ],
        out_shape=[
            jax.ShapeDtypeStruct((CHUNK_E[kc], D), jnp.float32),
            jax.ShapeDtypeStruct((gec, WPR, 128), jnp.float32),
        ],
    )(g, edge_h, edge_qrh, edge_qeh, wmk, wqt)


def _scatter_body(metas, wmsg_hbms, w_hbms, dst_hbm, nump_hbm, denp_hbm,
                  ix0, ix1, wm0, wm1, wv0, wv1, zb_v, zb1_v, num_sh, den_sh,
                  lsem0, lsem1, ssem0, ssem1):
    c = lax.axis_index("c")
    s = lax.axis_index("s")
    wid = s * NC + c
    ix = (ix0, ix1)
    wm = (wm0, wm1)
    wv = (wv0, wv1)
    lsem = (lsem0, lsem1)
    ssem = (ssem0, ssem1)

    subs = []
    for si, (eb, npw, nch) in enumerate(metas):
        for j in range(nch):
            subs.append((si, eb, npw, j))
    nsub = len(subs)

    def srcs(t):
        si, eb, npw, j = subs[t]
        off = wid * npw + j * CH
        return wmsg_hbms[si].at[pl.ds(off, CH)], w_hbms[si].at[pl.ds(off, CH)], \
            dst_hbm.at[pl.ds(eb + off, CH)]

    def lstart(t, b):
        wm_s, w_s, d_s = srcs(t)
        pltpu.async_copy(d_s, ix[b], lsem[b])
        pltpu.async_copy(wm_s, wm[b], lsem[b])
        pltpu.async_copy(w_s, wv[b], lsem[b])

    def lwait(t, b):
        wm_s, w_s, d_s = srcs(t)
        pltpu.make_async_copy(d_s, ix[b], lsem[b]).wait()
        pltpu.make_async_copy(wm_s, wm[b], lsem[b]).wait()
        pltpu.make_async_copy(w_s, wv[b], lsem[b]).wait()

    lstart(0, 0)

    def zrow(i, carry):
        for l in range(D // 16):
            zb_v[i, pl.ds(l * 16, 16)] = jnp.zeros((16,), jnp.float32)
        return carry

    lax.fori_loop(0, ZR, zrow, 0)

    def zrow1(i, carry):
        zb1_v[pl.ds(i * 16, 16)] = jnp.zeros((16,), jnp.float32)
        return carry

    lax.fori_loop(0, 40, zrow1, 0)

    def zc(i, carry):
        cc = s + i * NS

        @pl.when(cc < NZCT)
        def _():
            pltpu.sync_copy(zb_v, num_sh.at[pl.ds(cc * ZR, ZR)])

        return carry

    lax.fori_loop(0, 16, zc, 0)
    pltpu.sync_copy(zb1_v, den_sh.at[pl.ds(s * 640, 640)])
    plsc.subcore_barrier()

    def sstart(j, b):
        pltpu.async_copy(wm[b], num_sh.at[ix[b]], ssem[b], add=True)
        pltpu.async_copy(wv[b], den_sh.at[ix[b]], ssem[b], add=True)

    def swait(b):
        pltpu.make_async_copy(wm[b], num_sh.at[pl.ds(0, CH)], ssem[b]).wait()
        pltpu.make_async_copy(wv[b], den_sh.at[pl.ds(0, CH)], ssem[b]).wait()

    for t in range(nsub):
        b = t & 1
        lwait(t, b)
        if t >= 1:
            swait(1 - b)
        if t + 1 < nsub:
            lstart(t + 1, 1 - b)
        sstart(t, b)
    swait((nsub - 1) & 1)
    plsc.subcore_barrier()

    def rc(i, carry):
        cc = s + i * NS

        @pl.when(cc < NZCT)
        def _():
            pltpu.sync_copy(num_sh.at[pl.ds(cc * ZR, ZR)], zb_v)
            pltpu.sync_copy(zb_v, nump_hbm.at[c, pl.ds(cc * ZR, ZR)])

        return carry

    lax.fori_loop(0, 16, rc, 0)
    pltpu.sync_copy(den_sh.at[pl.ds(s * 640, 640)], zb1_v)
    pltpu.sync_copy(zb1_v, denp_hbm.at[c, pl.ds(s * 640, 640)])


def _sc_scatter(wmsgs, ws, dst, group):
    metas = tuple((CHUNK_B[kc], CHUNK_E[kc] // NW, CHUNK_E[kc] // NW // CH)
                  for kc in group)
    ng = len(group)

    def body(*args):
        _scatter_body(metas, args[0:ng], args[ng:2 * ng], *args[2 * ng:])

    k = pl.kernel(
        body,
        out_type=(
            jax.ShapeDtypeStruct((NC, N, D), jnp.float32),
            jax.ShapeDtypeStruct((NC, NP1), jnp.float32),
        ),
        mesh=_mesh(),
        scratch_types=[
            pltpu.VMEM((CH,), jnp.int32),
            pltpu.VMEM((CH,), jnp.int32),
            pltpu.VMEM((CH, D), jnp.float32),
            pltpu.VMEM((CH, D), jnp.float32),
            pltpu.VMEM((CH,), jnp.float32),
            pltpu.VMEM((CH,), jnp.float32),
            pltpu.VMEM((ZR, D), jnp.float32),
            pltpu.VMEM((640,), jnp.float32),
            pltpu.VMEM_SHARED((N, D), jnp.float32),
            pltpu.VMEM_SHARED((NP1,), jnp.float32),
            pltpu.SemaphoreType.DMA,
            pltpu.SemaphoreType.DMA,
            pltpu.SemaphoreType.DMA,
            pltpu.SemaphoreType.DMA,
        ],
    )
    return k(*wmsgs, *ws, dst)


NB = 1000


def _final_body(np0, np1, np2, den_ref, x_ref, wout_ref, gamma_ref, beta_ref, out_ref):
    num = (np0[0] + np0[1] + np1[0] + np1[1] + np2[0] + np2[1])
    den = den_ref[...]
    agg = num * jnp.where(den > 0, 1.0 / jnp.where(den > 0, den, 1.0), 0.0)
    t = jnp.dot(agg, wout_ref[...], preferred_element_type=jnp.float32)
    h = jnp.where(t >= 0, t, 0.01 * t) + x_ref[...]
    mu = jnp.mean(h, axis=-1, keepdims=True)
    var = jnp.mean((h - mu) ** 2, axis=-1, keepdims=True)
    out_ref[...] = (h - mu) * lax.rsqrt(var + 1e-6) * gamma_ref[...] + beta_ref[...]


def _tc_final(numps, den, x, wout, gamma, beta):
    return pl.pallas_call(
        _final_body,
        grid=(N // NB,),
        in_specs=[
            pl.BlockSpec((NC, NB, D), lambda i: (0, i, 0)),
            pl.BlockSpec((NC, NB, D), lambda i: (0, i, 0)),
            pl.BlockSpec((NC, NB, D), lambda i: (0, i, 0)),
            pl.BlockSpec((NB, 1), lambda i: (i, 0)),
            pl.BlockSpec((NB, D), lambda i: (i, 0)),
            pl.BlockSpec((D, D), lambda i: (0, 0)),
            pl.BlockSpec((1, D), lambda i: (0, 0)),
            pl.BlockSpec((1, D), lambda i: (0, 0)),
        ],
        out_specs=pl.BlockSpec((NB, D), lambda i: (i, 0)),
        out_shape=jax.ShapeDtypeStruct((N, D), jnp.float32),
    )(*numps, den, x, wout, gamma, beta)


def kernel(x, edge_h, edge_qrh, edge_qeh, W_msg, W_q, W_k, W_out, ln_gamma, ln_beta, edge_index):
    src = edge_index[0].astype(jnp.int32)
    dst = edge_index[1].astype(jnp.int32)
    wmk = jnp.concatenate([W_msg.T, W_k.T], axis=1)
    wqt = W_q.T * (1.0 / TEMP)

    wmsgs, ws = [], []
    for kc in range(K):
        g = _sc_gather(x, src, kc)
        wmsg, wp = _tc_edge(g, edge_h, edge_qrh, edge_qeh, wmk, wqt, kc)
        wmsgs.append(wmsg)
        ws.append(wp.reshape(CHUNK_E[kc]))
    numps = []
    den = jnp.zeros((N,), jnp.float32)
    for group in ((0, 1), (2, 3), (4,)):
        nump, denp = _sc_scatter([wmsgs[kc] for kc in group],
                                 [ws[kc] for kc in group], dst, group)
        numps.append(nump)
        den = den + denp[0, :N] + denp[1, :N]
    return _tc_final(numps, den.reshape(N, 1), x, W_out.T,
                     ln_gamma.reshape(1, D), ln_beta.reshape(1, D))


# speedup vs baseline: 1.0199x; 1.0012x over previous
import functools

import jax
import jax.numpy as jnp
from jax import lax
from jax.experimental import pallas as pl
from jax.experimental.pallas import tpu as pltpu
from jax.experimental.pallas import tpu_sc as plsc

D = 128
N = 10000
E = 320000
TEMP = float(D) ** 0.5

NC = 2
NS = 16
NW = NC * NS
EPW = E // NW
CH = 80
NCH = EPW // CH

ZR = 40
NZCT = N // ZR
NP1 = 10240


def _mesh():
    return plsc.VectorSubcoreMesh(core_axis_name="c", subcore_axis_name="s")


K = 5
UNITS = [10, 30, 30, 30, 25]
CHUNK_E = [u * NW * CH for u in UNITS]
CHUNK_B = [NW * CH * sum(UNITS[:i]) for i in range(K)]


NBUF = 4


def _gather_body(eb, npw, nch, x_hbm, src_hbm, g_hbm, idx_v, *bufs):
    rows = bufs[0:NBUF]
    gsem = bufs[NBUF:2 * NBUF]
    wsem = bufs[2 * NBUF:3 * NBUF]
    c = lax.axis_index("c")
    s = lax.axis_index("s")
    wid = s * NC + c
    base = eb + wid * npw
    pltpu.sync_copy(src_hbm.at[pl.ds(base, npw)], idx_v)

    def gstart(j, b):
        pltpu.async_copy(x_hbm.at[idx_v.at[pl.ds(j * CH, CH)]], rows[b], gsem[b])

    def gwait(b):
        pltpu.make_async_copy(x_hbm.at[pl.ds(0, CH)], rows[b], gsem[b]).wait()

    def wstart(j, b):
        pltpu.async_copy(rows[b], g_hbm.at[pl.ds(wid * npw + j * CH, CH)], wsem[b])

    def wwait(j, b):
        pltpu.make_async_copy(rows[b], g_hbm.at[pl.ds(wid * npw + j * CH, CH)],
                              wsem[b]).wait()

    for j in range(min(NBUF - 1, nch)):
        gstart(j, j % NBUF)
    for j in range(nch):
        b = j % NBUF
        gwait(b)
        wstart(j, b)
        ahead = j + NBUF - 1
        if ahead < nch:
            ab = ahead % NBUF
            if ahead >= NBUF:
                wwait(ahead - NBUF, ab)
            gstart(ahead, ab)
    for j in range(max(0, nch - NBUF), nch):
        wwait(j, j % NBUF)


def _sc_gather(x, src, kc):
    ec = CHUNK_E[kc]
    npw = ec // NW
    k = pl.kernel(
        functools.partial(_gather_body, CHUNK_B[kc], npw, npw // CH),
        out_type=jax.ShapeDtypeStruct((ec, D), jnp.float32),
        mesh=_mesh(),
        scratch_types=(
            [pltpu.VMEM((npw,), jnp.int32)]
            + [pltpu.VMEM((CH, D), jnp.float32)] * NBUF
            + [pltpu.SemaphoreType.DMA] * (2 * NBUF)
        ),
    )
    return k(x, src)


EB = 1280
NEB = E // EB
WPR = EB // 128


def _edge_body(g_ref, eh_ref, qrh_ref, qeh_ref, wmk_ref, wqt_ref, wmsg_ref, wp_ref):
    g = g_ref[...]
    eh = eh_ref[...]
    mk = (jnp.dot(g, wmk_ref[:D], preferred_element_type=jnp.float32)
          + jnp.dot(eh, wmk_ref[D:], preferred_element_type=jnp.float32))
    q = (jnp.dot(qrh_ref[...], wqt_ref[:D], preferred_element_type=jnp.float32)
         + jnp.dot(qeh_ref[...], wqt_ref[D:], preferred_element_type=jnp.float32))
    m = mk[:, :D]
    msg = jnp.where(m >= 0, m, 0.01 * m)
    k = mk[:, D:]
    att = jnp.sum(q * k, axis=-1, keepdims=True)
    w = jnp.exp(att)
    wmsg_ref[...] = w * msg
    e_i = lax.broadcasted_iota(jnp.int32, (EB, 128), 0)
    l_i = lax.broadcasted_iota(jnp.int32, (EB, 128), 1)
    B = (e_i % 128 == l_i).astype(jnp.float32)
    g_i = lax.broadcasted_iota(jnp.int32, (WPR, EB), 0)
    e2_i = lax.broadcasted_iota(jnp.int32, (WPR, EB), 1)
    A = (e2_i // 128 == g_i).astype(jnp.float32)
    wp_ref[0] = jnp.dot(A, w * B, preferred_element_type=jnp.float32)


def _tc_edge(g, edge_h, edge_qrh, edge_qeh, wmk, wqt, kc):
    gec = CHUNK_E[kc] // EB
    off = CHUNK_B[kc] // EB
    return pl.pallas_call(
        _edge_body,
        grid=(gec,),
        in_specs=[
            pl.BlockSpec((EB, D), lambda i: (i, 0)),
            pl.BlockSpec((EB, D), lambda i: (i + off, 0)),
            pl.BlockSpec((EB, D), lambda i: (i + off, 0)),
            pl.BlockSpec((EB, D), lambda i: (i + off, 0)),
            pl.BlockSpec((2 * D, 2 * D), lambda i: (0, 0)),
            pl.BlockSpec((2 * D, D), lambda i: (0, 0)),
        ],
        out_specs=[
            pl.BlockSpec((EB, D), lambda i: (i, 0)),
            pl.BlockSpec((1, WPR, 128), lambda i: (i, 0, 0)),
        ],
        out_shape=[
            jax.ShapeDtypeStruct((CHUNK_E[kc], D), jnp.float32),
            jax.ShapeDtypeStruct((gec, WPR, 128), jnp.float32),
        ],
    )(g, edge_h, edge_qrh, edge_qeh, wmk, wqt)


def _scatter_body(metas, wmsg_hbms, w_hbms, dst_hbm, nump_hbm, denp_hbm,
                  zb_v, zb1_v, num_sh, den_sh, *bufs):
    ix = bufs[0:NBUF]
    wm = bufs[NBUF:2 * NBUF]
    wv = bufs[2 * NBUF:3 * NBUF]
    lsem = bufs[3 * NBUF:4 * NBUF]
    ssem = bufs[4 * NBUF:5 * NBUF]
    c = lax.axis_index("c")
    s = lax.axis_index("s")
    wid = s * NC + c

    subs = []
    for si, (eb, npw, nch) in enumerate(metas):
        for j in range(nch):
            subs.append((si, eb, npw, j))
    nsub = len(subs)

    def srcs(t):
        si, eb, npw, j = subs[t]
        off = wid * npw + j * CH
        return wmsg_hbms[si].at[pl.ds(off, CH)], w_hbms[si].at[pl.ds(off, CH)], \
            dst_hbm.at[pl.ds(eb + off, CH)]

    def lstart(t, b):
        wm_s, w_s, d_s = srcs(t)
        pltpu.async_copy(d_s, ix[b], lsem[b])
        pltpu.async_copy(wm_s, wm[b], lsem[b])
        pltpu.async_copy(w_s, wv[b], lsem[b])

    def lwait(t, b):
        wm_s, w_s, d_s = srcs(t)
        pltpu.make_async_copy(d_s, ix[b], lsem[b]).wait()
        pltpu.make_async_copy(wm_s, wm[b], lsem[b]).wait()
        pltpu.make_async_copy(w_s, wv[b], lsem[b]).wait()

    for t in range(min(NBUF - 1, nsub)):
        lstart(t, t % NBUF)

    def zrow(i, carry):
        for l in range(D // 16):
            zb_v[i, pl.ds(l * 16, 16)] = jnp.zeros((16,), jnp.float32)
        return carry

    lax.fori_loop(0, ZR, zrow, 0)

    def zrow1(i, carry):
        zb1_v[pl.ds(i * 16, 16)] = jnp.zeros((16,), jnp.float32)
        return carry

    lax.fori_loop(0, 40, zrow1, 0)

    def zc(i, carry):
        cc = s + i * NS

        @pl.when(cc < NZCT)
        def _():
            pltpu.sync_copy(zb_v, num_sh.at[pl.ds(cc * ZR, ZR)])

        return carry

    lax.fori_loop(0, 16, zc, 0)
    pltpu.sync_copy(zb1_v, den_sh.at[pl.ds(s * 640, 640)])
    plsc.subcore_barrier()

    def sstart(j, b):
        pltpu.async_copy(wm[b], num_sh.at[ix[b]], ssem[b], add=True)
        pltpu.async_copy(wv[b], den_sh.at[ix[b]], ssem[b], add=True)

    def swait(b):
        pltpu.make_async_copy(wm[b], num_sh.at[pl.ds(0, CH)], ssem[b]).wait()
        pltpu.make_async_copy(wv[b], den_sh.at[pl.ds(0, CH)], ssem[b]).wait()

    for t in range(nsub):
        b = t % NBUF
        lwait(t, b)
        ahead = t + NBUF - 1
        if ahead < nsub:
            ab = ahead % NBUF
            if ahead >= NBUF:
                swait(ab)
            lstart(ahead, ab)
        sstart(t, b)
    for t in range(max(0, nsub - NBUF), nsub):
        swait(t % NBUF)
    plsc.subcore_barrier()

    def rc(i, carry):
        cc = s + i * NS

        @pl.when(cc < NZCT)
        def _():
            pltpu.sync_copy(num_sh.at[pl.ds(cc * ZR, ZR)], zb_v)
            pltpu.sync_copy(zb_v, nump_hbm.at[c, pl.ds(cc * ZR, ZR)])

        return carry

    lax.fori_loop(0, 16, rc, 0)
    pltpu.sync_copy(den_sh.at[pl.ds(s * 640, 640)], zb1_v)
    pltpu.sync_copy(zb1_v, denp_hbm.at[c, pl.ds(s * 640, 640)])


def _sc_scatter(wmsgs, ws, dst, group):
    metas = tuple((CHUNK_B[kc], CHUNK_E[kc] // NW, CHUNK_E[kc] // NW // CH)
                  for kc in group)
    ng = len(group)

    def body(*args):
        _scatter_body(metas, args[0:ng], args[ng:2 * ng], *args[2 * ng:])

    k = pl.kernel(
        body,
        out_type=(
            jax.ShapeDtypeStruct((NC, N, D), jnp.float32),
            jax.ShapeDtypeStruct((NC, NP1), jnp.float32),
        ),
        mesh=_mesh(),
        scratch_types=(
            [pltpu.VMEM((ZR, D), jnp.float32),
             pltpu.VMEM((640,), jnp.float32),
             pltpu.VMEM_SHARED((N, D), jnp.float32),
             pltpu.VMEM_SHARED((NP1,), jnp.float32)]
            + [pltpu.VMEM((CH,), jnp.int32)] * NBUF
            + [pltpu.VMEM((CH, D), jnp.float32)] * NBUF
            + [pltpu.VMEM((CH,), jnp.float32)] * NBUF
            + [pltpu.SemaphoreType.DMA] * (2 * NBUF)
        ),
    )
    return k(*wmsgs, *ws, dst)


NB = 1000


def _final_body(np0, np1, np2, den_ref, x_ref, wout_ref, gamma_ref, beta_ref, out_ref):
    num = (np0[0] + np0[1] + np1[0] + np1[1] + np2[0] + np2[1])
    den = den_ref[...]
    agg = num * jnp.where(den > 0, 1.0 / jnp.where(den > 0, den, 1.0), 0.0)
    t = jnp.dot(agg, wout_ref[...], preferred_element_type=jnp.float32)
    h = jnp.where(t >= 0, t, 0.01 * t) + x_ref[...]
    mu = jnp.mean(h, axis=-1, keepdims=True)
    var = jnp.mean((h - mu) ** 2, axis=-1, keepdims=True)
    out_ref[...] = (h - mu) * lax.rsqrt(var + 1e-6) * gamma_ref[...] + beta_ref[...]


def _tc_final(numps, den, x, wout, gamma, beta):
    return pl.pallas_call(
        _final_body,
        grid=(N // NB,),
        in_specs=[
            pl.BlockSpec((NC, NB, D), lambda i: (0, i, 0)),
            pl.BlockSpec((NC, NB, D), lambda i: (0, i, 0)),
            pl.BlockSpec((NC, NB, D), lambda i: (0, i, 0)),
            pl.BlockSpec((NB, 1), lambda i: (i, 0)),
            pl.BlockSpec((NB, D), lambda i: (i, 0)),
            pl.BlockSpec((D, D), lambda i: (0, 0)),
            pl.BlockSpec((1, D), lambda i: (0, 0)),
            pl.BlockSpec((1, D), lambda i: (0, 0)),
        ],
        out_specs=pl.BlockSpec((NB, D), lambda i: (i, 0)),
        out_shape=jax.ShapeDtypeStruct((N, D), jnp.float32),
    )(*numps, den, x, wout, gamma, beta)


def kernel(x, edge_h, edge_qrh, edge_qeh, W_msg, W_q, W_k, W_out, ln_gamma, ln_beta, edge_index):
    src = edge_index[0].astype(jnp.int32)
    dst = edge_index[1].astype(jnp.int32)
    wmk = jnp.concatenate([W_msg.T, W_k.T], axis=1)
    wqt = W_q.T * (1.0 / TEMP)

    wmsgs, ws = [], []
    for kc in range(K):
        g = _sc_gather(x, src, kc)
        wmsg, wp = _tc_edge(g, edge_h, edge_qrh, edge_qeh, wmk, wqt, kc)
        wmsgs.append(wmsg)
        ws.append(wp.reshape(CHUNK_E[kc]))
    numps = []
    den = jnp.zeros((N,), jnp.float32)
    for group in ((0, 1), (2, 3), (4,)):
        nump, denp = _sc_scatter([wmsgs[kc] for kc in group],
                                 [ws[kc] for kc in group], dst, group)
        numps.append(nump)
        den = den + denp[0, :N] + denp[1, :N]
    return _tc_final(numps, den.reshape(N, 1), x, W_out.T,
                     ln_gamma.reshape(1, D), ln_beta.reshape(1, D))


# speedup vs baseline: 1.0333x; 1.0131x over previous
import functools

import jax
import jax.numpy as jnp
from jax import lax
from jax.experimental import pallas as pl
from jax.experimental.pallas import tpu as pltpu
from jax.experimental.pallas import tpu_sc as plsc

D = 128
N = 10000
E = 320000
TEMP = float(D) ** 0.5

NC = 2
NS = 16
NW = NC * NS
EPW = E // NW
CH = 80
NCH = EPW // CH

ZR = 40
NZCT = N // ZR
NP1 = 10240


def _mesh():
    return plsc.VectorSubcoreMesh(core_axis_name="c", subcore_axis_name="s")


K = 5
UNITS = [10, 30, 35, 40, 10]
CHUNK_E = [u * NW * CH for u in UNITS]
CHUNK_B = [NW * CH * sum(UNITS[:i]) for i in range(K)]


NBUF = 4


def _gather_body(eb, npw, nch, x_hbm, src_hbm, g_hbm, idx_v, *bufs):
    rows = bufs[0:NBUF]
    gsem = bufs[NBUF:2 * NBUF]
    wsem = bufs[2 * NBUF:3 * NBUF]
    c = lax.axis_index("c")
    s = lax.axis_index("s")
    wid = s * NC + c
    base = eb + wid * npw
    pltpu.sync_copy(src_hbm.at[pl.ds(base, npw)], idx_v)

    def gstart(j, b):
        pltpu.async_copy(x_hbm.at[idx_v.at[pl.ds(j * CH, CH)]], rows[b], gsem[b])

    def gwait(b):
        pltpu.make_async_copy(x_hbm.at[pl.ds(0, CH)], rows[b], gsem[b]).wait()

    def wstart(j, b):
        pltpu.async_copy(rows[b], g_hbm.at[pl.ds(wid * npw + j * CH, CH)], wsem[b])

    def wwait(j, b):
        pltpu.make_async_copy(rows[b], g_hbm.at[pl.ds(wid * npw + j * CH, CH)],
                              wsem[b]).wait()

    for j in range(min(NBUF - 1, nch)):
        gstart(j, j % NBUF)
    for j in range(nch):
        b = j % NBUF
        gwait(b)
        wstart(j, b)
        ahead = j + NBUF - 1
        if ahead < nch:
            ab = ahead % NBUF
            if ahead >= NBUF:
                wwait(ahead - NBUF, ab)
            gstart(ahead, ab)
    for j in range(max(0, nch - NBUF), nch):
        wwait(j, j % NBUF)


def _sc_gather(x, src, kc):
    ec = CHUNK_E[kc]
    npw = ec // NW
    k = pl.kernel(
        functools.partial(_gather_body, CHUNK_B[kc], npw, npw // CH),
        out_type=jax.ShapeDtypeStruct((ec, D), jnp.float32),
        mesh=_mesh(),
        scratch_types=(
            [pltpu.VMEM((npw,), jnp.int32)]
            + [pltpu.VMEM((CH, D), jnp.float32)] * NBUF
            + [pltpu.SemaphoreType.DMA] * (2 * NBUF)
        ),
    )
    return k(x, src)


EB = 1280
NEB = E // EB
WPR = EB // 128


def _edge_body(g_ref, eh_ref, qrh_ref, qeh_ref, wmk_ref, wqt_ref, wmsg_ref, wp_ref):
    g = g_ref[...]
    eh = eh_ref[...]
    mk = (jnp.dot(g, wmk_ref[:D], preferred_element_type=jnp.float32)
          + jnp.dot(eh, wmk_ref[D:], preferred_element_type=jnp.float32))
    q = (jnp.dot(qrh_ref[...], wqt_ref[:D], preferred_element_type=jnp.float32)
         + jnp.dot(qeh_ref[...], wqt_ref[D:], preferred_element_type=jnp.float32))
    m = mk[:, :D]
    msg = jnp.where(m >= 0, m, 0.01 * m)
    k = mk[:, D:]
    att = jnp.sum(q * k, axis=-1, keepdims=True)
    w = jnp.exp(att)
    wmsg_ref[...] = w * msg
    e_i = lax.broadcasted_iota(jnp.int32, (EB, 128), 0)
    l_i = lax.broadcasted_iota(jnp.int32, (EB, 128), 1)
    B = (e_i % 128 == l_i).astype(jnp.float32)
    g_i = lax.broadcasted_iota(jnp.int32, (WPR, EB), 0)
    e2_i = lax.broadcasted_iota(jnp.int32, (WPR, EB), 1)
    A = (e2_i // 128 == g_i).astype(jnp.float32)
    wp_ref[0] = jnp.dot(A, w * B, preferred_element_type=jnp.float32)


def _tc_edge(g, edge_h, edge_qrh, edge_qeh, wmk, wqt, kc):
    gec = CHUNK_E[kc] // EB
    off = CHUNK_B[kc] // EB
    return pl.pallas_call(
        _edge_body,
        grid=(gec,),
        in_specs=[
            pl.BlockSpec((EB, D), lambda i: (i, 0)),
            pl.BlockSpec((EB, D), lambda i: (i + off, 0)),
            pl.BlockSpec((EB, D), lambda i: (i + off, 0)),
            pl.BlockSpec((EB, D), lambda i: (i + off, 0)),
            pl.BlockSpec((2 * D, 2 * D), lambda i: (0, 0)),
            pl.BlockSpec((2 * D, D), lambda i: (0, 0)),
        ],
        out_specs=[
            pl.BlockSpec((EB, D), lambda i: (i, 0)),
            pl.BlockSpec((1, WPR, 128), lambda i: (i, 0, 0)),
        ],
        out_shape=[
            jax.ShapeDtypeStruct((CHUNK_E[kc], D), jnp.float32),
            jax.ShapeDtypeStruct((gec, WPR, 128), jnp.float32),
        ],
    )(g, edge_h, edge_qrh, edge_qeh, wmk, wqt)


def _scatter_body(metas, wmsg_hbms, w_hbms, dst_hbm, nump_hbm, denp_hbm,
                  zb_v, zb1_v, num_sh, den_sh, *bufs):
    ix = bufs[0:NBUF]
    wm = bufs[NBUF:2 * NBUF]
    wv = bufs[2 * NBUF:3 * NBUF]
    lsem = bufs[3 * NBUF:4 * NBUF]
    ssem = bufs[4 * NBUF:5 * NBUF]
    zsem = ssem[0]
    rsem = (ssem[0], ssem[1])
    c = lax.axis_index("c")
    s = lax.axis_index("s")
    wid = s * NC + c

    subs = []
    for si, (eb, npw, nch) in enumerate(metas):
        for j in range(nch):
            subs.append((si, eb, npw, j))
    nsub = len(subs)

    def srcs(t):
        si, eb, npw, j = subs[t]
        off = wid * npw + j * CH
        return wmsg_hbms[si].at[pl.ds(off, CH)], w_hbms[si].at[pl.ds(off, CH)], \
            dst_hbm.at[pl.ds(eb + off, CH)]

    def lstart(t, b):
        wm_s, w_s, d_s = srcs(t)
        pltpu.async_copy(d_s, ix[b], lsem[b])
        pltpu.async_copy(wm_s, wm[b], lsem[b])
        pltpu.async_copy(w_s, wv[b], lsem[b])

    def lwait(t, b):
        wm_s, w_s, d_s = srcs(t)
        pltpu.make_async_copy(d_s, ix[b], lsem[b]).wait()
        pltpu.make_async_copy(wm_s, wm[b], lsem[b]).wait()
        pltpu.make_async_copy(w_s, wv[b], lsem[b]).wait()

    for t in range(min(NBUF - 1, nsub)):
        lstart(t, t % NBUF)

    def zrow(i, carry):
        for l in range(D // 16):
            zb_v[i, pl.ds(l * 16, 16)] = jnp.zeros((16,), jnp.float32)
        return carry

    lax.fori_loop(0, ZR, zrow, 0)

    def zrow1(i, carry):
        zb1_v[pl.ds(i * 16, 16)] = jnp.zeros((16,), jnp.float32)
        return carry

    lax.fori_loop(0, 40, zrow1, 0)

    for i in range(16):
        cc = s + i * NS

        @pl.when(cc < NZCT)
        def _(cc=cc):
            pltpu.async_copy(zb_v, num_sh.at[pl.ds(cc * ZR, ZR)], zsem)

    pltpu.async_copy(zb1_v, den_sh.at[pl.ds(s * 640, 640)], zsem)
    for i in range(16):
        cc = s + i * NS

        @pl.when(cc < NZCT)
        def _(cc=cc):
            pltpu.make_async_copy(zb_v, num_sh.at[pl.ds(cc * ZR, ZR)], zsem).wait()

    pltpu.make_async_copy(zb1_v, den_sh.at[pl.ds(s * 640, 640)], zsem).wait()
    plsc.subcore_barrier()

    def sstart(j, b):
        pltpu.async_copy(wm[b], num_sh.at[ix[b]], ssem[b], add=True)
        pltpu.async_copy(wv[b], den_sh.at[ix[b]], ssem[b], add=True)

    def swait(b):
        pltpu.make_async_copy(wm[b], num_sh.at[pl.ds(0, CH)], ssem[b]).wait()
        pltpu.make_async_copy(wv[b], den_sh.at[pl.ds(0, CH)], ssem[b]).wait()

    for t in range(nsub):
        b = t % NBUF
        lwait(t, b)
        ahead = t + NBUF - 1
        if ahead < nsub:
            ab = ahead % NBUF
            if ahead >= NBUF:
                swait(ab)
            lstart(ahead, ab)
        sstart(t, b)
    for t in range(max(0, nsub - NBUF), nsub):
        swait(t % NBUF)
    plsc.subcore_barrier()

    zb = (zb_v, wm[0].at[pl.ds(0, ZR)])

    def rstore(cc, b):
        pltpu.async_copy(zb[b], nump_hbm.at[c, pl.ds(cc * ZR, ZR)], rsem[b])

    def rwait(cc, b):
        pltpu.make_async_copy(zb[b], nump_hbm.at[c, pl.ds(cc * ZR, ZR)],
                              rsem[b]).wait()

    for i in range(16):
        cc = s + i * NS
        b = i & 1

        @pl.when(cc < NZCT)
        def _(cc=cc, b=b, i=i):
            if i >= 2:
                rwait(cc - 2 * NS, b)
            pltpu.sync_copy(num_sh.at[pl.ds(cc * ZR, ZR)], zb[b])
            rstore(cc, b)

    pltpu.sync_copy(den_sh.at[pl.ds(s * 640, 640)], zb1_v)
    pltpu.sync_copy(zb1_v, denp_hbm.at[c, pl.ds(s * 640, 640)])
    rwait(s + 14 * NS, 0)

    @pl.when(s + 15 * NS < NZCT)
    def _():
        rwait(s + 15 * NS, 1)

    @pl.when(s + 15 * NS >= NZCT)
    def _():
        rwait(s + 13 * NS, 1)


def _sc_scatter(wmsgs, ws, dst, group):
    metas = tuple((CHUNK_B[kc], CHUNK_E[kc] // NW, CHUNK_E[kc] // NW // CH)
                  for kc in group)
    ng = len(group)

    def body(*args):
        _scatter_body(metas, args[0:ng], args[ng:2 * ng], *args[2 * ng:])

    k = pl.kernel(
        body,
        out_type=(
            jax.ShapeDtypeStruct((NC, N, D), jnp.float32),
            jax.ShapeDtypeStruct((NC, NP1), jnp.float32),
        ),
        mesh=_mesh(),
        scratch_types=(
            [pltpu.VMEM((ZR, D), jnp.float32),
             pltpu.VMEM((640,), jnp.float32),
             pltpu.VMEM_SHARED((N, D), jnp.float32),
             pltpu.VMEM_SHARED((NP1,), jnp.float32)]
            + [pltpu.VMEM((CH,), jnp.int32)] * NBUF
            + [pltpu.VMEM((CH, D), jnp.float32)] * NBUF
            + [pltpu.VMEM((CH,), jnp.float32)] * NBUF
            + [pltpu.SemaphoreType.DMA] * (2 * NBUF)
        ),
    )
    return k(*wmsgs, *ws, dst)


NB = 1000


def _final_body(np0, np1, np2, den_ref, x_ref, wout_ref, gamma_ref, beta_ref, out_ref):
    num = (np0[0] + np0[1] + np1[0] + np1[1] + np2[0] + np2[1])
    den = den_ref[...]
    agg = num * jnp.where(den > 0, 1.0 / jnp.where(den > 0, den, 1.0), 0.0)
    t = jnp.dot(agg, wout_ref[...], preferred_element_type=jnp.float32)
    h = jnp.where(t >= 0, t, 0.01 * t) + x_ref[...]
    mu = jnp.mean(h, axis=-1, keepdims=True)
    var = jnp.mean((h - mu) ** 2, axis=-1, keepdims=True)
    out_ref[...] = (h - mu) * lax.rsqrt(var + 1e-6) * gamma_ref[...] + beta_ref[...]


def _tc_final(numps, den, x, wout, gamma, beta):
    return pl.pallas_call(
        _final_body,
        grid=(N // NB,),
        in_specs=[
            pl.BlockSpec((NC, NB, D), lambda i: (0, i, 0)),
            pl.BlockSpec((NC, NB, D), lambda i: (0, i, 0)),
            pl.BlockSpec((NC, NB, D), lambda i: (0, i, 0)),
            pl.BlockSpec((NB, 1), lambda i: (i, 0)),
            pl.BlockSpec((NB, D), lambda i: (i, 0)),
            pl.BlockSpec((D, D), lambda i: (0, 0)),
            pl.BlockSpec((1, D), lambda i: (0, 0)),
            pl.BlockSpec((1, D), lambda i: (0, 0)),
        ],
        out_specs=pl.BlockSpec((NB, D), lambda i: (i, 0)),
        out_shape=jax.ShapeDtypeStruct((N, D), jnp.float32),
    )(*numps, den, x, wout, gamma, beta)


def kernel(x, edge_h, edge_qrh, edge_qeh, W_msg, W_q, W_k, W_out, ln_gamma, ln_beta, edge_index):
    src = edge_index[0].astype(jnp.int32)
    dst = edge_index[1].astype(jnp.int32)
    wmk = jnp.concatenate([W_msg.T, W_k.T], axis=1)
    wqt = W_q.T * (1.0 / TEMP)

    wmsgs, ws = [], []
    for kc in range(K):
        g = _sc_gather(x, src, kc)
        wmsg, wp = _tc_edge(g, edge_h, edge_qrh, edge_qeh, wmk, wqt, kc)
        wmsgs.append(wmsg)
        ws.append(wp.reshape(CHUNK_E[kc]))
    numps = []
    den = jnp.zeros((N,), jnp.float32)
    for group in ((0, 1), (2, 3), (4,)):
        nump, denp = _sc_scatter([wmsgs[kc] for kc in group],
                                 [ws[kc] for kc in group], dst, group)
        numps.append(nump)
        den = den + denp[0, :N] + denp[1, :N]
    return _tc_final(numps, den.reshape(N, 1), x, W_out.T,
                     ln_gamma.reshape(1, D), ln_beta.reshape(1, D))


# speedup vs baseline: 1.1254x; 1.0892x over previous
import functools

import jax
import jax.numpy as jnp
from jax import lax
from jax.experimental import pallas as pl
from jax.experimental.pallas import tpu as pltpu
from jax.experimental.pallas import tpu_sc as plsc

D = 128
N = 10000
E = 320000
TEMP = float(D) ** 0.5

NC = 2
NS = 16
NW = NC * NS
EPW = E // NW
CH = 80
NCH = EPW // CH

ZR = 40
NZCT = N // ZR
NP1 = 10240


def _mesh():
    return plsc.VectorSubcoreMesh(core_axis_name="c", subcore_axis_name="s")


K = 5
UNITS = [10, 30, 35, 40, 10]
CHUNK_E = [u * NW * CH for u in UNITS]
CHUNK_B = [NW * CH * sum(UNITS[:i]) for i in range(K)]


NBUF = 4


def _gather_body(eb, npw, nch, x_hbm, src_hbm, g_hbm, idx_v, *bufs):
    rows = bufs[0:NBUF]
    gsem = bufs[NBUF:2 * NBUF]
    wsem = bufs[2 * NBUF:3 * NBUF]
    c = lax.axis_index("c")
    s = lax.axis_index("s")
    wid = s * NC + c
    base = eb + wid * npw
    pltpu.sync_copy(src_hbm.at[pl.ds(base, npw)], idx_v)

    def gstart(j, b):
        pltpu.async_copy(x_hbm.at[idx_v.at[pl.ds(j * CH, CH)]], rows[b], gsem[b])

    def gwait(b):
        pltpu.make_async_copy(x_hbm.at[pl.ds(0, CH)], rows[b], gsem[b]).wait()

    def wstart(j, b):
        pltpu.async_copy(rows[b], g_hbm.at[pl.ds(wid * npw + j * CH, CH)], wsem[b])

    def wwait(j, b):
        pltpu.make_async_copy(rows[b], g_hbm.at[pl.ds(wid * npw + j * CH, CH)],
                              wsem[b]).wait()

    for j in range(min(NBUF - 1, nch)):
        gstart(j, j % NBUF)
    for j in range(nch):
        b = j % NBUF
        gwait(b)
        wstart(j, b)
        ahead = j + NBUF - 1
        if ahead < nch:
            ab = ahead % NBUF
            if ahead >= NBUF:
                wwait(ahead - NBUF, ab)
            gstart(ahead, ab)
    for j in range(max(0, nch - NBUF), nch):
        wwait(j, j % NBUF)


def _sc_gather(x, src, kc):
    ec = CHUNK_E[kc]
    npw = ec // NW
    k = pl.kernel(
        functools.partial(_gather_body, CHUNK_B[kc], npw, npw // CH),
        out_type=jax.ShapeDtypeStruct((ec, D), jnp.float32),
        mesh=_mesh(),
        scratch_types=(
            [pltpu.VMEM((npw,), jnp.int32)]
            + [pltpu.VMEM((CH, D), jnp.float32)] * NBUF
            + [pltpu.SemaphoreType.DMA] * (2 * NBUF)
        ),
    )
    return k(x, src)


EB = 2560
NEB = E // EB
WPR = EB // 128


def _edge_body(g_ref, eh_ref, qrh_ref, qeh_ref, wmk_ref, wqt_ref, wmsg_ref, wp_ref):
    g = g_ref[...]
    eh = eh_ref[...]
    mk = (jnp.dot(g, wmk_ref[:D], preferred_element_type=jnp.float32)
          + jnp.dot(eh, wmk_ref[D:], preferred_element_type=jnp.float32))
    q = (jnp.dot(qrh_ref[...], wqt_ref[:D], preferred_element_type=jnp.float32)
         + jnp.dot(qeh_ref[...], wqt_ref[D:], preferred_element_type=jnp.float32))
    m = mk[:, :D]
    msg = jnp.where(m >= 0, m, 0.01 * m)
    k = mk[:, D:]
    att = jnp.sum(q * k, axis=-1, keepdims=True)
    w = jnp.exp(att)
    wmsg_ref[...] = w * msg
    e_i = lax.broadcasted_iota(jnp.int32, (EB, 128), 0)
    l_i = lax.broadcasted_iota(jnp.int32, (EB, 128), 1)
    B = (e_i % 128 == l_i).astype(jnp.float32)
    g_i = lax.broadcasted_iota(jnp.int32, (WPR, EB), 0)
    e2_i = lax.broadcasted_iota(jnp.int32, (WPR, EB), 1)
    A = (e2_i // 128 == g_i).astype(jnp.float32)
    wp_ref[0] = jnp.dot(A, w * B, preferred_element_type=jnp.float32)


def _tc_edge(g, edge_h, edge_qrh, edge_qeh, wmk, wqt, kc):
    gec = CHUNK_E[kc] // EB
    off = CHUNK_B[kc] // EB
    return pl.pallas_call(
        _edge_body,
        grid=(gec,),
        in_specs=[
            pl.BlockSpec((EB, D), lambda i: (i, 0)),
            pl.BlockSpec((EB, D), lambda i: (i + off, 0)),
            pl.BlockSpec((EB, D), lambda i: (i + off, 0)),
            pl.BlockSpec((EB, D), lambda i: (i + off, 0)),
            pl.BlockSpec((2 * D, 2 * D), lambda i: (0, 0)),
            pl.BlockSpec((2 * D, D), lambda i: (0, 0)),
        ],
        out_specs=[
            pl.BlockSpec((EB, D), lambda i: (i, 0)),
            pl.BlockSpec((1, WPR, 128), lambda i: (i, 0, 0)),
        ],
        out_shape=[
            jax.ShapeDtypeStruct((CHUNK_E[kc], D), jnp.float32),
            jax.ShapeDtypeStruct((gec, WPR, 128), jnp.float32),
        ],
    )(g, edge_h, edge_qrh, edge_qeh, wmk, wqt)


def _scatter_body(metas, wmsg_hbms, w_hbms, dst_hbm, nump_hbm, denp_hbm,
                  zb_v, zb1_v, num_sh, den_sh, *bufs):
    ix = bufs[0:NBUF]
    wm = bufs[NBUF:2 * NBUF]
    wv = bufs[2 * NBUF:3 * NBUF]
    lsem = bufs[3 * NBUF:4 * NBUF]
    ssem = bufs[4 * NBUF:5 * NBUF]
    zsem = ssem[0]
    rsem = (ssem[0], ssem[1])
    c = lax.axis_index("c")
    s = lax.axis_index("s")
    wid = s * NC + c

    subs = []
    for si, (eb, npw, nch) in enumerate(metas):
        for j in range(nch):
            subs.append((si, eb, npw, j))
    nsub = len(subs)

    def srcs(t):
        si, eb, npw, j = subs[t]
        off = wid * npw + j * CH
        return wmsg_hbms[si].at[pl.ds(off, CH)], w_hbms[si].at[pl.ds(off, CH)], \
            dst_hbm.at[pl.ds(eb + off, CH)]

    def lstart(t, b):
        wm_s, w_s, d_s = srcs(t)
        pltpu.async_copy(d_s, ix[b], lsem[b])
        pltpu.async_copy(wm_s, wm[b], lsem[b])
        pltpu.async_copy(w_s, wv[b], lsem[b])

    def lwait(t, b):
        wm_s, w_s, d_s = srcs(t)
        pltpu.make_async_copy(d_s, ix[b], lsem[b]).wait()
        pltpu.make_async_copy(wm_s, wm[b], lsem[b]).wait()
        pltpu.make_async_copy(w_s, wv[b], lsem[b]).wait()

    for t in range(min(NBUF - 1, nsub)):
        lstart(t, t % NBUF)

    def zrow(i, carry):
        for l in range(D // 16):
            zb_v[i, pl.ds(l * 16, 16)] = jnp.zeros((16,), jnp.float32)
        return carry

    lax.fori_loop(0, ZR, zrow, 0)

    def zrow1(i, carry):
        zb1_v[pl.ds(i * 16, 16)] = jnp.zeros((16,), jnp.float32)
        return carry

    lax.fori_loop(0, 40, zrow1, 0)

    for i in range(16):
        cc = s + i * NS

        @pl.when(cc < NZCT)
        def _(cc=cc):
            pltpu.async_copy(zb_v, num_sh.at[pl.ds(cc * ZR, ZR)], zsem)

    pltpu.async_copy(zb1_v, den_sh.at[pl.ds(s * 640, 640)], zsem)
    for i in range(16):
        cc = s + i * NS

        @pl.when(cc < NZCT)
        def _(cc=cc):
            pltpu.make_async_copy(zb_v, num_sh.at[pl.ds(cc * ZR, ZR)], zsem).wait()

    pltpu.make_async_copy(zb1_v, den_sh.at[pl.ds(s * 640, 640)], zsem).wait()
    plsc.subcore_barrier()

    def sstart(j, b):
        pltpu.async_copy(wm[b], num_sh.at[ix[b]], ssem[b], add=True)
        pltpu.async_copy(wv[b], den_sh.at[ix[b]], ssem[b], add=True)

    def swait(b):
        pltpu.make_async_copy(wm[b], num_sh.at[pl.ds(0, CH)], ssem[b]).wait()
        pltpu.make_async_copy(wv[b], den_sh.at[pl.ds(0, CH)], ssem[b]).wait()

    for t in range(nsub):
        b = t % NBUF
        lwait(t, b)
        ahead = t + NBUF - 1
        if ahead < nsub:
            ab = ahead % NBUF
            if ahead >= NBUF:
                swait(ab)
            lstart(ahead, ab)
        sstart(t, b)
    for t in range(max(0, nsub - NBUF), nsub):
        swait(t % NBUF)
    plsc.subcore_barrier()

    zb = (zb_v, wm[0].at[pl.ds(0, ZR)])

    def rstore(cc, b):
        pltpu.async_copy(zb[b], nump_hbm.at[c, pl.ds(cc * ZR, ZR)], rsem[b])

    def rwait(cc, b):
        pltpu.make_async_copy(zb[b], nump_hbm.at[c, pl.ds(cc * ZR, ZR)],
                              rsem[b]).wait()

    for i in range(16):
        cc = s + i * NS
        b = i & 1

        @pl.when(cc < NZCT)
        def _(cc=cc, b=b, i=i):
            if i >= 2:
                rwait(cc - 2 * NS, b)
            pltpu.sync_copy(num_sh.at[pl.ds(cc * ZR, ZR)], zb[b])
            rstore(cc, b)

    pltpu.sync_copy(den_sh.at[pl.ds(s * 640, 640)], zb1_v)
    pltpu.sync_copy(zb1_v, denp_hbm.at[c, pl.ds(s * 640, 640)])
    rwait(s + 14 * NS, 0)

    @pl.when(s + 15 * NS < NZCT)
    def _():
        rwait(s + 15 * NS, 1)

    @pl.when(s + 15 * NS >= NZCT)
    def _():
        rwait(s + 13 * NS, 1)


def _sc_scatter(wmsgs, ws, dst, group):
    metas = tuple((CHUNK_B[kc], CHUNK_E[kc] // NW, CHUNK_E[kc] // NW // CH)
                  for kc in group)
    ng = len(group)

    def body(*args):
        _scatter_body(metas, args[0:ng], args[ng:2 * ng], *args[2 * ng:])

    k = pl.kernel(
        body,
        out_type=(
            jax.ShapeDtypeStruct((NC, N, D), jnp.float32),
            jax.ShapeDtypeStruct((NC, NP1), jnp.float32),
        ),
        mesh=_mesh(),
        scratch_types=(
            [pltpu.VMEM((ZR, D), jnp.float32),
             pltpu.VMEM((640,), jnp.float32),
             pltpu.VMEM_SHARED((N, D), jnp.float32),
             pltpu.VMEM_SHARED((NP1,), jnp.float32)]
            + [pltpu.VMEM((CH,), jnp.int32)] * NBUF
            + [pltpu.VMEM((CH, D), jnp.float32)] * NBUF
            + [pltpu.VMEM((CH,), jnp.float32)] * NBUF
            + [pltpu.SemaphoreType.DMA] * (2 * NBUF)
        ),
    )
    return k(*wmsgs, *ws, dst)


NB = 1000


def _final_body(np0, np1, np2, den_ref, x_ref, wout_ref, gamma_ref, beta_ref, out_ref):
    num = (np0[0] + np0[1] + np1[0] + np1[1] + np2[0] + np2[1])
    den = den_ref[...]
    agg = num * jnp.where(den > 0, 1.0 / jnp.where(den > 0, den, 1.0), 0.0)
    t = jnp.dot(agg, wout_ref[...], preferred_element_type=jnp.float32)
    h = jnp.where(t >= 0, t, 0.01 * t) + x_ref[...]
    mu = jnp.mean(h, axis=-1, keepdims=True)
    var = jnp.mean((h - mu) ** 2, axis=-1, keepdims=True)
    out_ref[...] = (h - mu) * lax.rsqrt(var + 1e-6) * gamma_ref[...] + beta_ref[...]


def _tc_final(numps, den, x, wout, gamma, beta):
    return pl.pallas_call(
        _final_body,
        grid=(N // NB,),
        in_specs=[
            pl.BlockSpec((NC, NB, D), lambda i: (0, i, 0)),
            pl.BlockSpec((NC, NB, D), lambda i: (0, i, 0)),
            pl.BlockSpec((NC, NB, D), lambda i: (0, i, 0)),
            pl.BlockSpec((NB, 1), lambda i: (i, 0)),
            pl.BlockSpec((NB, D), lambda i: (i, 0)),
            pl.BlockSpec((D, D), lambda i: (0, 0)),
            pl.BlockSpec((1, D), lambda i: (0, 0)),
            pl.BlockSpec((1, D), lambda i: (0, 0)),
        ],
        out_specs=pl.BlockSpec((NB, D), lambda i: (i, 0)),
        out_shape=jax.ShapeDtypeStruct((N, D), jnp.float32),
    )(*numps, den, x, wout, gamma, beta)


def kernel(x, edge_h, edge_qrh, edge_qeh, W_msg, W_q, W_k, W_out, ln_gamma, ln_beta, edge_index):
    src = edge_index[0].astype(jnp.int32)
    dst = edge_index[1].astype(jnp.int32)
    wmk = jnp.concatenate([W_msg.T, W_k.T], axis=1)
    wqt = W_q.T * (1.0 / TEMP)

    wmsgs, ws = [], []
    for kc in range(K):
        g = _sc_gather(x, src, kc)
        wmsg, wp = _tc_edge(g, edge_h, edge_qrh, edge_qeh, wmk, wqt, kc)
        wmsgs.append(wmsg)
        ws.append(wp.reshape(CHUNK_E[kc]))
    numps = []
    den = jnp.zeros((N,), jnp.float32)
    for group in ((0, 1), (2, 3), (4,)):
        nump, denp = _sc_scatter([wmsgs[kc] for kc in group],
                                 [ws[kc] for kc in group], dst, group)
        numps.append(nump)
        den = den + denp[0, :N] + denp[1, :N]
    return _tc_final(numps, den.reshape(N, 1), x, W_out.T,
                     ln_gamma.reshape(1, D), ln_beta.reshape(1, D))


# speedup vs baseline: 1.1749x; 1.0440x over previous
import functools

import jax
import jax.numpy as jnp
from jax import lax
from jax.experimental import pallas as pl
from jax.experimental.pallas import tpu as pltpu
from jax.experimental.pallas import tpu_sc as plsc

D = 128
N = 10000
E = 320000
TEMP = float(D) ** 0.5

NC = 2
NS = 16
NW = NC * NS
EPW = E // NW
CH = 80
NCH = EPW // CH

ZR = 40
NZCT = N // ZR
NP1 = 10240


def _mesh():
    return plsc.VectorSubcoreMesh(core_axis_name="c", subcore_axis_name="s")


K = 5
UNITS = [10, 30, 36, 40, 9]
CHUNK_E = [u * NW * CH for u in UNITS]
CHUNK_B = [NW * CH * sum(UNITS[:i]) for i in range(K)]


NBUF = 4


def _gather_body(eb, npw, nch, x_hbm, src_hbm, g_hbm, idx_v, *bufs):
    rows = bufs[0:NBUF]
    gsem = bufs[NBUF:2 * NBUF]
    wsem = bufs[2 * NBUF:3 * NBUF]
    c = lax.axis_index("c")
    s = lax.axis_index("s")
    wid = s * NC + c
    base = eb + wid * npw
    pltpu.sync_copy(src_hbm.at[pl.ds(base, npw)], idx_v)

    def gstart(j, b):
        pltpu.async_copy(x_hbm.at[idx_v.at[pl.ds(j * CH, CH)]], rows[b], gsem[b])

    def gwait(b):
        pltpu.make_async_copy(x_hbm.at[pl.ds(0, CH)], rows[b], gsem[b]).wait()

    def wstart(j, b):
        pltpu.async_copy(rows[b], g_hbm.at[pl.ds(wid * npw + j * CH, CH)], wsem[b])

    def wwait(j, b):
        pltpu.make_async_copy(rows[b], g_hbm.at[pl.ds(wid * npw + j * CH, CH)],
                              wsem[b]).wait()

    for j in range(min(NBUF - 1, nch)):
        gstart(j, j % NBUF)
    for j in range(nch):
        b = j % NBUF
        gwait(b)
        wstart(j, b)
        ahead = j + NBUF - 1
        if ahead < nch:
            ab = ahead % NBUF
            if ahead >= NBUF:
                wwait(ahead - NBUF, ab)
            gstart(ahead, ab)
    for j in range(max(0, nch - NBUF), nch):
        wwait(j, j % NBUF)


def _sc_gather(x, src, kc):
    ec = CHUNK_E[kc]
    npw = ec // NW
    k = pl.kernel(
        functools.partial(_gather_body, CHUNK_B[kc], npw, npw // CH),
        out_type=jax.ShapeDtypeStruct((ec, D), jnp.float32),
        mesh=_mesh(),
        scratch_types=(
            [pltpu.VMEM((npw,), jnp.int32)]
            + [pltpu.VMEM((CH, D), jnp.float32)] * NBUF
            + [pltpu.SemaphoreType.DMA] * (2 * NBUF)
        ),
    )
    return k(x, src)


def _edge_body(EB, WPR, g_ref, eh_ref, qrh_ref, qeh_ref, wmk_ref, wqt_ref, wmsg_ref, wp_ref):
    g = g_ref[...]
    eh = eh_ref[...]
    mk = (jnp.dot(g, wmk_ref[:D], preferred_element_type=jnp.float32)
          + jnp.dot(eh, wmk_ref[D:], preferred_element_type=jnp.float32))
    q = (jnp.dot(qrh_ref[...], wqt_ref[:D], preferred_element_type=jnp.float32)
         + jnp.dot(qeh_ref[...], wqt_ref[D:], preferred_element_type=jnp.float32))
    m = mk[:, :D]
    msg = jnp.where(m >= 0, m, 0.01 * m)
    k = mk[:, D:]
    att = jnp.sum(q * k, axis=-1, keepdims=True)
    w = jnp.exp(att)
    wmsg_ref[...] = w * msg
    e_i = lax.broadcasted_iota(jnp.int32, (EB, 128), 0)
    l_i = lax.broadcasted_iota(jnp.int32, (EB, 128), 1)
    B = (e_i % 128 == l_i).astype(jnp.float32)
    g_i = lax.broadcasted_iota(jnp.int32, (WPR, EB), 0)
    e2_i = lax.broadcasted_iota(jnp.int32, (WPR, EB), 1)
    A = (e2_i // 128 == g_i).astype(jnp.float32)
    wp_ref[0] = jnp.dot(A, w * B, preferred_element_type=jnp.float32)


def _tc_edge(g, edge_h, edge_qrh, edge_qeh, wmk, wqt, kc):
    EB = 5120 if CHUNK_E[kc] % 5120 == 0 else 2560
    WPR = EB // 128
    gec = CHUNK_E[kc] // EB
    off = CHUNK_B[kc] // EB
    return pl.pallas_call(
        functools.partial(_edge_body, EB, WPR),
        grid=(gec,),
        in_specs=[
            pl.BlockSpec((EB, D), lambda i: (i, 0)),
            pl.BlockSpec((EB, D), lambda i: (i + off, 0)),
            pl.BlockSpec((EB, D), lambda i: (i + off, 0)),
            pl.BlockSpec((EB, D), lambda i: (i + off, 0)),
            pl.BlockSpec((2 * D, 2 * D), lambda i: (0, 0)),
            pl.BlockSpec((2 * D, D), lambda i: (0, 0)),
        ],
        out_specs=[
            pl.BlockSpec((EB, D), lambda i: (i, 0)),
            pl.BlockSpec((1, WPR, 128), lambda i: (i, 0, 0)),
        ],
        out_shape=[
            jax.ShapeDtypeStruct((CHUNK_E[kc], D), jnp.float32),
            jax.ShapeDtypeStruct((gec, WPR, 128), jnp.float32),
        ],
    )(g, edge_h, edge_qrh, edge_qeh, wmk, wqt)


def _scatter_body(metas, wmsg_hbms, w_hbms, dst_hbm, nump_hbm, denp_hbm,
                  zb_v, zb1_v, num_sh, den_sh, *bufs):
    ix = bufs[0:NBUF]
    wm = bufs[NBUF:2 * NBUF]
    wv = bufs[2 * NBUF:3 * NBUF]
    lsem = bufs[3 * NBUF:4 * NBUF]
    ssem = bufs[4 * NBUF:5 * NBUF]
    zsem = ssem[0]
    rsem = (ssem[0], ssem[1])
    c = lax.axis_index("c")
    s = lax.axis_index("s")
    wid = s * NC + c

    subs = []
    for si, (eb, npw, nch) in enumerate(metas):
        for j in range(nch):
            subs.append((si, eb, npw, j))
    nsub = len(subs)

    def srcs(t):
        si, eb, npw, j = subs[t]
        off = wid * npw + j * CH
        return wmsg_hbms[si].at[pl.ds(off, CH)], w_hbms[si].at[pl.ds(off, CH)], \
            dst_hbm.at[pl.ds(eb + off, CH)]

    def lstart(t, b):
        wm_s, w_s, d_s = srcs(t)
        pltpu.async_copy(d_s, ix[b], lsem[b])
        pltpu.async_copy(wm_s, wm[b], lsem[b])
        pltpu.async_copy(w_s, wv[b], lsem[b])

    def lwait(t, b):
        wm_s, w_s, d_s = srcs(t)
        pltpu.make_async_copy(d_s, ix[b], lsem[b]).wait()
        pltpu.make_async_copy(wm_s, wm[b], lsem[b]).wait()
        pltpu.make_async_copy(w_s, wv[b], lsem[b]).wait()

    for t in range(min(NBUF - 1, nsub)):
        lstart(t, t % NBUF)

    def zrow(i, carry):
        for l in range(D // 16):
            zb_v[i, pl.ds(l * 16, 16)] = jnp.zeros((16,), jnp.float32)
        return carry

    lax.fori_loop(0, ZR, zrow, 0)

    def zrow1(i, carry):
        zb1_v[pl.ds(i * 16, 16)] = jnp.zeros((16,), jnp.float32)
        return carry

    lax.fori_loop(0, 40, zrow1, 0)

    for i in range(16):
        cc = s + i * NS

        @pl.when(cc < NZCT)
        def _(cc=cc):
            pltpu.async_copy(zb_v, num_sh.at[pl.ds(cc * ZR, ZR)], zsem)

    pltpu.async_copy(zb1_v, den_sh.at[pl.ds(s * 640, 640)], zsem)
    for i in range(16):
        cc = s + i * NS

        @pl.when(cc < NZCT)
        def _(cc=cc):
            pltpu.make_async_copy(zb_v, num_sh.at[pl.ds(cc * ZR, ZR)], zsem).wait()

    pltpu.make_async_copy(zb1_v, den_sh.at[pl.ds(s * 640, 640)], zsem).wait()
    plsc.subcore_barrier()

    def sstart(j, b):
        pltpu.async_copy(wm[b], num_sh.at[ix[b]], ssem[b], add=True)
        pltpu.async_copy(wv[b], den_sh.at[ix[b]], ssem[b], add=True)

    def swait(b):
        pltpu.make_async_copy(wm[b], num_sh.at[pl.ds(0, CH)], ssem[b]).wait()
        pltpu.make_async_copy(wv[b], den_sh.at[pl.ds(0, CH)], ssem[b]).wait()

    for t in range(nsub):
        b = t % NBUF
        lwait(t, b)
        ahead = t + NBUF - 1
        if ahead < nsub:
            ab = ahead % NBUF
            if ahead >= NBUF:
                swait(ab)
            lstart(ahead, ab)
        sstart(t, b)
    for t in range(max(0, nsub - NBUF), nsub):
        swait(t % NBUF)
    plsc.subcore_barrier()

    zb = (zb_v, wm[0].at[pl.ds(0, ZR)])

    def rstore(cc, b):
        pltpu.async_copy(zb[b], nump_hbm.at[c, pl.ds(cc * ZR, ZR)], rsem[b])

    def rwait(cc, b):
        pltpu.make_async_copy(zb[b], nump_hbm.at[c, pl.ds(cc * ZR, ZR)],
                              rsem[b]).wait()

    for i in range(16):
        cc = s + i * NS
        b = i & 1

        @pl.when(cc < NZCT)
        def _(cc=cc, b=b, i=i):
            if i >= 2:
                rwait(cc - 2 * NS, b)
            pltpu.sync_copy(num_sh.at[pl.ds(cc * ZR, ZR)], zb[b])
            rstore(cc, b)

    pltpu.sync_copy(den_sh.at[pl.ds(s * 640, 640)], zb1_v)
    pltpu.sync_copy(zb1_v, denp_hbm.at[c, pl.ds(s * 640, 640)])
    rwait(s + 14 * NS, 0)

    @pl.when(s + 15 * NS < NZCT)
    def _():
        rwait(s + 15 * NS, 1)

    @pl.when(s + 15 * NS >= NZCT)
    def _():
        rwait(s + 13 * NS, 1)


def _sc_scatter(wmsgs, ws, dst, group):
    metas = tuple((CHUNK_B[kc], CHUNK_E[kc] // NW, CHUNK_E[kc] // NW // CH)
                  for kc in group)
    ng = len(group)

    def body(*args):
        _scatter_body(metas, args[0:ng], args[ng:2 * ng], *args[2 * ng:])

    k = pl.kernel(
        body,
        out_type=(
            jax.ShapeDtypeStruct((NC, N, D), jnp.float32),
            jax.ShapeDtypeStruct((NC, NP1), jnp.float32),
        ),
        mesh=_mesh(),
        scratch_types=(
            [pltpu.VMEM((ZR, D), jnp.float32),
             pltpu.VMEM((640,), jnp.float32),
             pltpu.VMEM_SHARED((N, D), jnp.float32),
             pltpu.VMEM_SHARED((NP1,), jnp.float32)]
            + [pltpu.VMEM((CH,), jnp.int32)] * NBUF
            + [pltpu.VMEM((CH, D), jnp.float32)] * NBUF
            + [pltpu.VMEM((CH,), jnp.float32)] * NBUF
            + [pltpu.SemaphoreType.DMA] * (2 * NBUF)
        ),
    )
    return k(*wmsgs, *ws, dst)


NB = 1000


def _final_body(np0, np1, np2, den_ref, x_ref, wout_ref, gamma_ref, beta_ref, out_ref):
    num = (np0[0] + np0[1] + np1[0] + np1[1] + np2[0] + np2[1])
    den = den_ref[...]
    agg = num * jnp.where(den > 0, 1.0 / jnp.where(den > 0, den, 1.0), 0.0)
    t = jnp.dot(agg, wout_ref[...], preferred_element_type=jnp.float32)
    h = jnp.where(t >= 0, t, 0.01 * t) + x_ref[...]
    mu = jnp.mean(h, axis=-1, keepdims=True)
    var = jnp.mean((h - mu) ** 2, axis=-1, keepdims=True)
    out_ref[...] = (h - mu) * lax.rsqrt(var + 1e-6) * gamma_ref[...] + beta_ref[...]


def _tc_final(numps, den, x, wout, gamma, beta):
    return pl.pallas_call(
        _final_body,
        grid=(N // NB,),
        in_specs=[
            pl.BlockSpec((NC, NB, D), lambda i: (0, i, 0)),
            pl.BlockSpec((NC, NB, D), lambda i: (0, i, 0)),
            pl.BlockSpec((NC, NB, D), lambda i: (0, i, 0)),
            pl.BlockSpec((NB, 1), lambda i: (i, 0)),
            pl.BlockSpec((NB, D), lambda i: (i, 0)),
            pl.BlockSpec((D, D), lambda i: (0, 0)),
            pl.BlockSpec((1, D), lambda i: (0, 0)),
            pl.BlockSpec((1, D), lambda i: (0, 0)),
        ],
        out_specs=pl.BlockSpec((NB, D), lambda i: (i, 0)),
        out_shape=jax.ShapeDtypeStruct((N, D), jnp.float32),
    )(*numps, den, x, wout, gamma, beta)


def kernel(x, edge_h, edge_qrh, edge_qeh, W_msg, W_q, W_k, W_out, ln_gamma, ln_beta, edge_index):
    src = edge_index[0].astype(jnp.int32)
    dst = edge_index[1].astype(jnp.int32)
    wmk = jnp.concatenate([W_msg.T, W_k.T], axis=1)
    wqt = W_q.T * (1.0 / TEMP)

    wmsgs, ws = [], []
    for kc in range(K):
        g = _sc_gather(x, src, kc)
        wmsg, wp = _tc_edge(g, edge_h, edge_qrh, edge_qeh, wmk, wqt, kc)
        wmsgs.append(wmsg)
        ws.append(wp.reshape(CHUNK_E[kc]))
    numps = []
    den = jnp.zeros((N,), jnp.float32)
    for group in ((0, 1), (2, 3), (4,)):
        nump, denp = _sc_scatter([wmsgs[kc] for kc in group],
                                 [ws[kc] for kc in group], dst, group)
        numps.append(nump)
        den = den + denp[0, :N] + denp[1, :N]
    return _tc_final(numps, den.reshape(N, 1), x, W_out.T,
                     ln_gamma.reshape(1, D), ln_beta.reshape(1, D))
